# baseline (speedup 1.0000x reference)
import functools

import jax
import jax.numpy as jnp
from jax import lax
from jax.experimental import pallas as pl
from jax.experimental.pallas import tpu as pltpu
from jax.experimental.pallas import tpu_sc as plsc

N = 8192
K = 8192
C = 256
NB = 512
KB = 4096


def _argmin_body(zsq_ref, csq_ref, z_ref, cb_ref, out_ref, mmin_ref, midx_ref):
    j = pl.program_id(1)
    dot = lax.dot_general(z_ref[...], cb_ref[...],
                          (((1,), (1,)), ((), ())),
                          preferred_element_type=jnp.float32)
    t1 = zsq_ref[...] + csq_ref[...]
    d = jnp.sqrt(jnp.maximum(t1 - 2.0 * dot, 0.0))
    m = jnp.min(d, axis=1, keepdims=True)
    col = lax.broadcasted_iota(jnp.int32, (NB, KB), 1) + j * KB
    idx = jnp.min(jnp.where(d == m, col, jnp.int32(K)), axis=1, keepdims=True)
    mb = m.astype(jnp.bfloat16).astype(jnp.float32)

    @pl.when(j == 0)
    def _():
        mmin_ref[...] = mb
        midx_ref[...] = idx

    @pl.when(j > 0)
    def _():
        better = m < mmin_ref[...]
        midx_ref[...] = jnp.where(better, idx, midx_ref[...])
        mmin_ref[...] = jnp.where(better, mb, mmin_ref[...])

    @pl.when(j == pl.num_programs(1) - 1)
    def _():
        out_ref[...] = midx_ref[...]


_argmin_call = pl.pallas_call(
    _argmin_body,
    grid=(N // NB, K // KB),
    in_specs=[
        pl.BlockSpec((NB, 1), lambda i, j: (i, 0)),
        pl.BlockSpec((1, KB), lambda i, j: (0, j)),
        pl.BlockSpec((NB, C), lambda i, j: (i, 0)),
        pl.BlockSpec((KB, C), lambda i, j: (j, 0)),
    ],
    out_specs=pl.BlockSpec((NB, 1), lambda i, j: (i, 0)),
    out_shape=jax.ShapeDtypeStruct((N, 1), jnp.int32),
    scratch_shapes=[
        pltpu.VMEM((NB, 1), jnp.float32),
        pltpu.VMEM((NB, 1), jnp.int32),
    ],
    compiler_params=pltpu.CompilerParams(
        dimension_semantics=("parallel", "arbitrary")),
)

CHUNK = 128


@functools.cache
def _make_sc_gather():
    info = plsc.get_sparse_core_info()
    nc, ns = info.num_cores, info.num_subcores
    nw = nc * ns
    bpw = N // nw
    nch = bpw // CHUNK
    mesh = plsc.VectorSubcoreMesh(core_axis_name="c", subcore_axis_name="s")

    @functools.partial(
        pl.kernel,
        out_type=jax.ShapeDtypeStruct((N, C), jnp.float32),
        mesh=mesh,
        scratch_types=[
            pltpu.VMEM((nch, CHUNK), jnp.int32),
            pltpu.VMEM((bpw, C), jnp.float32),
            pltpu.SemaphoreType.DMA,
        ],
    )
    def _sc_gather(table_hbm, idx_hbm, out_hbm, idx_v, rows_v, sem):
        wid = lax.axis_index("s") * nc + lax.axis_index("c")
        pltpu.sync_copy(idx_hbm.at[pl.ds(wid * nch, nch)], idx_v)
        for jj in range(nch):
            pltpu.async_copy(table_hbm.at[idx_v.at[jj]],
                             rows_v.at[pl.ds(jj * CHUNK, CHUNK)], sem).wait()
        pltpu.sync_copy(rows_v, out_hbm.at[pl.ds(wid * bpw, bpw)])

    return _sc_gather


def kernel(z, codebook):
    B, L, C_ = z.shape
    z_e = z.reshape(B * L, C_)
    zsq = jnp.sum(z_e * z_e, axis=1, keepdims=True)
    csq = jnp.sum(codebook * codebook, axis=1)[None, :]
    idx = _argmin_call(zsq, csq, z_e, codebook).reshape(-1)
    z_q = _make_sc_gather()(codebook, idx.reshape(N // CHUNK, CHUNK))
    z_q_st = z_e + lax.stop_gradient(z_q - z_e)
    return (z_q_st, z_e, idx)


# speedup vs baseline: 1.0032x; 1.0032x over previous
import functools

import jax
import jax.numpy as jnp
from jax import lax
from jax.experimental import pallas as pl
from jax.experimental.pallas import tpu as pltpu
from jax.experimental.pallas import tpu_sc as plsc

N = 8192
K = 8192
C = 256
NB = 512
KB = 4096


def _argmin_body(zsq_ref, csq_ref, z_ref, cb_ref, out_ref, mmin_ref, midx_ref):
    j = pl.program_id(1)
    dot = lax.dot_general(z_ref[...], cb_ref[...],
                          (((1,), (1,)), ((), ())),
                          preferred_element_type=jnp.float32)
    t1 = zsq_ref[...] + csq_ref[...]
    d = jnp.sqrt(jnp.maximum(t1 - 2.0 * dot, 0.0))
    m = jnp.min(d, axis=1, keepdims=True)
    db = lax.bitcast_convert_type(d, jnp.int32)
    mb_i = lax.bitcast_convert_type(m, jnp.int32)
    col = lax.broadcasted_iota(jnp.int32, (NB, KB), 1)
    idx = jnp.min(jnp.where(db == mb_i, col, jnp.int32(K)), axis=1,
                  keepdims=True) + j * KB
    mb = m.astype(jnp.bfloat16).astype(jnp.float32)

    @pl.when(j == 0)
    def _():
        mmin_ref[...] = mb
        midx_ref[...] = idx

    @pl.when(j > 0)
    def _():
        better = m < mmin_ref[...]
        midx_ref[...] = jnp.where(better, idx, midx_ref[...])
        mmin_ref[...] = jnp.where(better, mb, mmin_ref[...])

    @pl.when(j == pl.num_programs(1) - 1)
    def _():
        out_ref[...] = midx_ref[...]


_argmin_call = pl.pallas_call(
    _argmin_body,
    grid=(N // NB, K // KB),
    in_specs=[
        pl.BlockSpec((NB, 1), lambda i, j: (i, 0)),
        pl.BlockSpec((1, KB), lambda i, j: (0, j)),
        pl.BlockSpec((NB, C), lambda i, j: (i, 0)),
        pl.BlockSpec((KB, C), lambda i, j: (j, 0)),
    ],
    out_specs=pl.BlockSpec((NB, 1), lambda i, j: (i, 0)),
    out_shape=jax.ShapeDtypeStruct((N, 1), jnp.int32),
    scratch_shapes=[
        pltpu.VMEM((NB, 1), jnp.float32),
        pltpu.VMEM((NB, 1), jnp.int32),
    ],
    compiler_params=pltpu.CompilerParams(
        dimension_semantics=("parallel", "arbitrary")),
)

CHUNK = 128


@functools.cache
def _make_sc_gather():
    info = plsc.get_sparse_core_info()
    nc, ns = info.num_cores, info.num_subcores
    nw = nc * ns
    bpw = N // nw
    nch = bpw // CHUNK
    mesh = plsc.VectorSubcoreMesh(core_axis_name="c", subcore_axis_name="s")

    @functools.partial(
        pl.kernel,
        out_type=jax.ShapeDtypeStruct((N, C), jnp.float32),
        mesh=mesh,
        scratch_types=[
            pltpu.VMEM((nch, CHUNK), jnp.int32),
            pltpu.VMEM((bpw, C), jnp.float32),
            pltpu.SemaphoreType.DMA,
        ],
    )
    def _sc_gather(table_hbm, idx_hbm, out_hbm, idx_v, rows_v, sem):
        wid = lax.axis_index("s") * nc + lax.axis_index("c")
        pltpu.sync_copy(idx_hbm.at[pl.ds(wid * nch, nch)], idx_v)
        for jj in range(nch):
            pltpu.async_copy(table_hbm.at[idx_v.at[jj]],
                             rows_v.at[pl.ds(jj * CHUNK, CHUNK)], sem).wait()
        pltpu.sync_copy(rows_v, out_hbm.at[pl.ds(wid * bpw, bpw)])

    return _sc_gather


def kernel(z, codebook):
    B, L, C_ = z.shape
    z_e = z.reshape(B * L, C_)
    zsq = jnp.sum(z_e * z_e, axis=1, keepdims=True)
    csq = jnp.sum(codebook * codebook, axis=1)[None, :]
    idx = _argmin_call(zsq, csq, z_e, codebook).reshape(-1)
    z_q = _make_sc_gather()(codebook, idx.reshape(N // CHUNK, CHUNK))
    z_q_st = z_e + lax.stop_gradient(z_q - z_e)
    return (z_q_st, z_e, idx)


# speedup vs baseline: 1.0353x; 1.0321x over previous
import functools

import jax
import jax.numpy as jnp
from jax import lax
from jax.experimental import pallas as pl
from jax.experimental.pallas import tpu as pltpu
from jax.experimental.pallas import tpu_sc as plsc

N = 8192
K = 8192
C = 256
NB = 512
KB = 4096


def _argmin_body(zsq_ref, csq_ref, z_ref, cb_ref, out_ref, mmin_ref, midx_ref):
    j = pl.program_id(1)
    dot = lax.dot_general(z_ref[...], cb_ref[...],
                          (((1,), (1,)), ((), ())),
                          preferred_element_type=jnp.float32)
    t1 = zsq_ref[...] + csq_ref[...]
    d = jnp.sqrt(jnp.maximum(t1 - dot, 0.0))
    m = jnp.min(d, axis=1, keepdims=True)
    db = lax.bitcast_convert_type(d, jnp.int32)
    mb_i = lax.bitcast_convert_type(m, jnp.int32)
    col = lax.broadcasted_iota(jnp.int32, (NB, KB), 1)
    idx = jnp.min(jnp.where(db == mb_i, col, jnp.int32(K)), axis=1,
                  keepdims=True) + j * KB
    mb = m.astype(jnp.bfloat16).astype(jnp.float32)

    @pl.when(j == 0)
    def _():
        mmin_ref[...] = mb
        midx_ref[...] = idx

    @pl.when(j > 0)
    def _():
        better = m < mmin_ref[...]
        midx_ref[...] = jnp.where(better, idx, midx_ref[...])
        mmin_ref[...] = jnp.where(better, mb, mmin_ref[...])

    @pl.when(j == pl.num_programs(1) - 1)
    def _():
        out_ref[...] = midx_ref[...]


_argmin_call = pl.pallas_call(
    _argmin_body,
    grid=(N // NB, K // KB),
    in_specs=[
        pl.BlockSpec((NB, 1), lambda i, j: (i, 0)),
        pl.BlockSpec((1, KB), lambda i, j: (0, j)),
        pl.BlockSpec((NB, C), lambda i, j: (i, 0)),
        pl.BlockSpec((KB, C), lambda i, j: (j, 0)),
    ],
    out_specs=pl.BlockSpec((NB, 1), lambda i, j: (i, 0)),
    out_shape=jax.ShapeDtypeStruct((N, 1), jnp.int32),
    scratch_shapes=[
        pltpu.VMEM((NB, 1), jnp.float32),
        pltpu.VMEM((NB, 1), jnp.int32),
    ],
    compiler_params=pltpu.CompilerParams(
        dimension_semantics=("parallel", "arbitrary")),
)

CHUNK = 128


@functools.cache
def _make_sc_gather():
    info = plsc.get_sparse_core_info()
    nc, ns = info.num_cores, info.num_subcores
    nw = nc * ns
    bpw = N // nw
    nch = bpw // CHUNK
    mesh = plsc.VectorSubcoreMesh(core_axis_name="c", subcore_axis_name="s")

    @functools.partial(
        pl.kernel,
        out_type=jax.ShapeDtypeStruct((N, C), jnp.float32),
        mesh=mesh,
        scratch_types=[
            pltpu.VMEM((nch, CHUNK), jnp.int32),
            pltpu.VMEM((bpw, C), jnp.float32),
            pltpu.SemaphoreType.DMA,
        ],
    )
    def _sc_gather(table_hbm, idx_hbm, out_hbm, idx_v, rows_v, sem):
        wid = lax.axis_index("s") * nc + lax.axis_index("c")
        pltpu.sync_copy(idx_hbm.at[pl.ds(wid * nch, nch)], idx_v)
        for jj in range(nch):
            pltpu.async_copy(table_hbm.at[idx_v.at[jj]],
                             rows_v.at[pl.ds(jj * CHUNK, CHUNK)], sem).wait()
        pltpu.sync_copy(rows_v, out_hbm.at[pl.ds(wid * bpw, bpw)])

    return _sc_gather


def kernel(z, codebook):
    B, L, C_ = z.shape
    z_e = z.reshape(B * L, C_)
    zsq = jnp.sum(z_e * z_e, axis=1, keepdims=True)
    csq = jnp.sum(codebook * codebook, axis=1)[None, :]
    idx = _argmin_call(zsq, csq, z_e, 2.0 * codebook).reshape(-1)
    z_q = _make_sc_gather()(codebook, idx.reshape(N // CHUNK, CHUNK))
    z_q_st = z_e + lax.stop_gradient(z_q - z_e)
    return (z_q_st, z_e, idx)


# speedup vs baseline: 1.1456x; 1.1066x over previous
import functools

import jax
import jax.numpy as jnp
from jax import lax
from jax.experimental import pallas as pl
from jax.experimental.pallas import tpu as pltpu
from jax.experimental.pallas import tpu_sc as plsc

N = 8192
K = 8192
C = 256
NB = 1024
KB = 4096


def _argmin_body(zsq_ref, csq_ref, z_ref, cb_ref, out_ref, mmin_ref, midx_ref):
    j = pl.program_id(1)
    dot = lax.dot_general(z_ref[...], cb_ref[...],
                          (((1,), (1,)), ((), ())),
                          preferred_element_type=jnp.float32)
    t1 = zsq_ref[...] + csq_ref[...]
    d = jnp.sqrt(jnp.maximum(t1 - dot, 0.0))
    m = jnp.min(d, axis=1, keepdims=True)
    db = lax.bitcast_convert_type(d, jnp.int32)
    mb_i = lax.bitcast_convert_type(m, jnp.int32)
    col = lax.broadcasted_iota(jnp.int32, (NB, KB), 1)
    idx = jnp.min(jnp.where(db == mb_i, col, jnp.int32(K)), axis=1,
                  keepdims=True) + j * KB
    mb = m.astype(jnp.bfloat16).astype(jnp.float32)

    @pl.when(j == 0)
    def _():
        mmin_ref[...] = mb
        midx_ref[...] = idx

    @pl.when(j > 0)
    def _():
        better = m < mmin_ref[...]
        midx_ref[...] = jnp.where(better, idx, midx_ref[...])
        mmin_ref[...] = jnp.where(better, mb, mmin_ref[...])

    @pl.when(j == pl.num_programs(1) - 1)
    def _():
        out_ref[...] = midx_ref[...]


_argmin_call = pl.pallas_call(
    _argmin_body,
    grid=(N // NB, K // KB),
    in_specs=[
        pl.BlockSpec((NB, 1), lambda i, j: (i, 0)),
        pl.BlockSpec((1, KB), lambda i, j: (0, j)),
        pl.BlockSpec((NB, C), lambda i, j: (i, 0)),
        pl.BlockSpec((KB, C), lambda i, j: (j, 0)),
    ],
    out_specs=pl.BlockSpec((NB, 1), lambda i, j: (i, 0)),
    out_shape=jax.ShapeDtypeStruct((N, 1), jnp.int32),
    scratch_shapes=[
        pltpu.VMEM((NB, 1), jnp.float32),
        pltpu.VMEM((NB, 1), jnp.int32),
    ],
    compiler_params=pltpu.CompilerParams(
        dimension_semantics=("parallel", "arbitrary")),
)

CHUNK = 128


@functools.cache
def _make_sc_gather():
    info = plsc.get_sparse_core_info()
    nc, ns = info.num_cores, info.num_subcores
    nw = nc * ns
    bpw = N // nw
    nch = bpw // CHUNK
    mesh = plsc.VectorSubcoreMesh(core_axis_name="c", subcore_axis_name="s")

    @functools.partial(
        pl.kernel,
        out_type=jax.ShapeDtypeStruct((N, C), jnp.float32),
        mesh=mesh,
        scratch_types=[
            pltpu.VMEM((nch, CHUNK), jnp.int32),
            pltpu.VMEM((bpw, C), jnp.float32),
            pltpu.SemaphoreType.DMA,
        ],
    )
    def _sc_gather(table_hbm, idx_hbm, out_hbm, idx_v, rows_v, sem):
        wid = lax.axis_index("s") * nc + lax.axis_index("c")
        pltpu.sync_copy(idx_hbm.at[pl.ds(wid * nch, nch)], idx_v)
        for jj in range(nch):
            pltpu.async_copy(table_hbm.at[idx_v.at[jj]],
                             rows_v.at[pl.ds(jj * CHUNK, CHUNK)], sem).wait()
        pltpu.sync_copy(rows_v, out_hbm.at[pl.ds(wid * bpw, bpw)])

    return _sc_gather


def kernel(z, codebook):
    B, L, C_ = z.shape
    z_e = z.reshape(B * L, C_)
    zsq = jnp.sum(z_e * z_e, axis=1, keepdims=True)
    csq = jnp.sum(codebook * codebook, axis=1)[None, :]
    idx = _argmin_call(zsq, csq, z_e, 2.0 * codebook).reshape(-1)
    z_q = _make_sc_gather()(codebook, idx.reshape(N // CHUNK, CHUNK))
    z_q_st = z_e + lax.stop_gradient(z_q - z_e)
    return (z_q_st, z_e, idx)


# speedup vs baseline: 1.1851x; 1.0344x over previous
import functools

import jax
import jax.numpy as jnp
from jax import lax
from jax.experimental import pallas as pl
from jax.experimental.pallas import tpu as pltpu
from jax.experimental.pallas import tpu_sc as plsc

N = 8192
K = 8192
C = 256
NB = 2048
KB = 4096


def _argmin_body(zsq_ref, csq_ref, z_ref, cb_ref, out_ref, mmin_ref, midx_ref):
    j = pl.program_id(1)
    dot = lax.dot_general(z_ref[...], cb_ref[...],
                          (((1,), (1,)), ((), ())),
                          preferred_element_type=jnp.float32)
    t1 = zsq_ref[...] + csq_ref[...]
    d = jnp.sqrt(jnp.maximum(t1 - dot, 0.0))
    m = jnp.min(d, axis=1, keepdims=True)
    db = lax.bitcast_convert_type(d, jnp.int32)
    mb_i = lax.bitcast_convert_type(m, jnp.int32)
    col = lax.broadcasted_iota(jnp.int32, (NB, KB), 1)
    idx = jnp.min(jnp.where(db == mb_i, col, jnp.int32(K)), axis=1,
                  keepdims=True) + j * KB
    mb = m.astype(jnp.bfloat16).astype(jnp.float32)

    @pl.when(j == 0)
    def _():
        mmin_ref[...] = mb
        midx_ref[...] = idx

    @pl.when(j > 0)
    def _():
        better = m < mmin_ref[...]
        midx_ref[...] = jnp.where(better, idx, midx_ref[...])
        mmin_ref[...] = jnp.where(better, mb, mmin_ref[...])

    @pl.when(j == pl.num_programs(1) - 1)
    def _():
        out_ref[...] = midx_ref[...]


_argmin_call = pl.pallas_call(
    _argmin_body,
    grid=(N // NB, K // KB),
    in_specs=[
        pl.BlockSpec((NB, 1), lambda i, j: (i, 0)),
        pl.BlockSpec((1, KB), lambda i, j: (0, j)),
        pl.BlockSpec((NB, C), lambda i, j: (i, 0)),
        pl.BlockSpec((KB, C), lambda i, j: (j, 0)),
    ],
    out_specs=pl.BlockSpec((NB, 1), lambda i, j: (i, 0)),
    out_shape=jax.ShapeDtypeStruct((N, 1), jnp.int32),
    scratch_shapes=[
        pltpu.VMEM((NB, 1), jnp.float32),
        pltpu.VMEM((NB, 1), jnp.int32),
    ],
    compiler_params=pltpu.CompilerParams(
        dimension_semantics=("parallel", "arbitrary")),
)

CHUNK = 128


@functools.cache
def _make_sc_gather():
    info = plsc.get_sparse_core_info()
    nc, ns = info.num_cores, info.num_subcores
    nw = nc * ns
    bpw = N // nw
    nch = bpw // CHUNK
    mesh = plsc.VectorSubcoreMesh(core_axis_name="c", subcore_axis_name="s")

    @functools.partial(
        pl.kernel,
        out_type=jax.ShapeDtypeStruct((N, C), jnp.float32),
        mesh=mesh,
        scratch_types=[
            pltpu.VMEM((nch, CHUNK), jnp.int32),
            pltpu.VMEM((bpw, C), jnp.float32),
            pltpu.SemaphoreType.DMA,
        ],
    )
    def _sc_gather(table_hbm, idx_hbm, out_hbm, idx_v, rows_v, sem):
        wid = lax.axis_index("s") * nc + lax.axis_index("c")
        pltpu.sync_copy(idx_hbm.at[pl.ds(wid * nch, nch)], idx_v)
        for jj in range(nch):
            pltpu.async_copy(table_hbm.at[idx_v.at[jj]],
                             rows_v.at[pl.ds(jj * CHUNK, CHUNK)], sem).wait()
        pltpu.sync_copy(rows_v, out_hbm.at[pl.ds(wid * bpw, bpw)])

    return _sc_gather


def kernel(z, codebook):
    B, L, C_ = z.shape
    z_e = z.reshape(B * L, C_)
    zsq = jnp.sum(z_e * z_e, axis=1, keepdims=True)
    csq = jnp.sum(codebook * codebook, axis=1)[None, :]
    idx = _argmin_call(zsq, csq, z_e, 2.0 * codebook).reshape(-1)
    z_q = _make_sc_gather()(codebook, idx.reshape(N // CHUNK, CHUNK))
    z_q_st = z_e + lax.stop_gradient(z_q - z_e)
    return (z_q_st, z_e, idx)
